# Initial kernel scaffold; baseline (speedup 1.0000x reference)
import jax
import jax.numpy as jnp
from jax.experimental import pallas as pl


def kernel(x, lut):
    raise NotImplementedError("write your pallas kernel here")


# speedup vs baseline: 2.4150x; 2.4150x over previous
import functools
import math

import jax
import jax.numpy as jnp
from jax import lax
from jax.experimental import pallas as pl
from jax.experimental.pallas import tpu as pltpu
from jax.experimental.pallas import tpu_sc as plsc

D_MODEL_K = 128
VOCAB_K = 100000
SCALE = math.sqrt(D_MODEL_K)

_info = plsc.get_sparse_core_info()
_NC, _NS, _L = _info.num_cores, _info.num_subcores, _info.num_lanes
_NW = _NC * _NS

_GROUP = 128


def _make_sc_gather(n_idx: int):
    assert n_idx % (_NW * _GROUP) == 0
    per_w = n_idx // _NW
    n_groups = per_w // _GROUP

    mesh = plsc.VectorSubcoreMesh(core_axis_name="c", subcore_axis_name="s")

    @functools.partial(
        pl.kernel,
        mesh=mesh,
        out_type=jax.ShapeDtypeStruct((n_idx, D_MODEL_K), jnp.float32),
        scratch_types=[
            pltpu.VMEM((n_groups, _GROUP), jnp.int32),
            pltpu.VMEM((_GROUP, D_MODEL_K), jnp.float32),
            pltpu.SemaphoreType.DMA,
        ],
    )
    def sc_gather(idx_hbm, table_hbm, out_hbm, idx_v, rows_v, sem):
        wid = lax.axis_index("s") * _NC + lax.axis_index("c")
        base = wid * per_w
        pltpu.sync_copy(idx_hbm.at[wid], idx_v)

        def group_body(g, carry):
            pltpu.async_copy(table_hbm.at[idx_v.at[g]], rows_v, sem).wait()

            def row_body(r, c2):
                for c in range(D_MODEL_K // _L):
                    sl = (r, pl.ds(c * _L, _L))
                    rows_v[sl] = rows_v[sl] * SCALE
                return c2

            lax.fori_loop(0, _GROUP, row_body, 0, unroll=False)
            pltpu.sync_copy(rows_v, out_hbm.at[pl.ds(base + g * _GROUP, _GROUP)])
            return carry

        lax.fori_loop(0, n_groups, group_body, 0, unroll=False)

    return sc_gather


def kernel(x, lut):
    b, s = x.shape
    n = b * s
    idx = x.reshape(_NW, n // (_NW * _GROUP), _GROUP).astype(jnp.int32)
    out = _make_sc_gather(n)(idx, lut)
    return out.reshape(b, s, D_MODEL_K)
